# baseline (speedup 1.0000x reference)
import functools

import jax
import jax.numpy as jnp
from jax import lax
from jax.experimental import pallas as pl
from jax.experimental.pallas import tpu as pltpu
from jax.experimental.pallas import tpu_sc as plsc

B, T, D = 4, 2048, 1024
NW = 32
TW = T // NW
R = 16
NC_CHUNKS = TW // R


def _sc_body(x_hbm, tab_hbm, out_hbm, tbuf, xbuf, obuf, sem):
    c = lax.axis_index("c")
    s = lax.axis_index("s")
    wid = s * 2 + c
    t_base = wid * TW

    def add_chunk(xb, tb, ob):
        def row(i, _):
            def col(j, _):
                off = pl.ds(j * 16, 16)
                ob[i, off] = xb[i, off] + tb[i, off]
                return 0
            lax.fori_loop(0, D // 16, col, 0)
            return 0
        lax.fori_loop(0, R, row, 0)

    for ci in range(NC_CHUNKS):
        t0 = t_base + ci * R
        pltpu.sync_copy(tab_hbm.at[pl.ds(t0, R)], tbuf)
        for b in range(B):
            pltpu.sync_copy(x_hbm.at[b, pl.ds(t0, R)], xbuf)
            add_chunk(xbuf, tbuf, obuf)
            pltpu.sync_copy(obuf, out_hbm.at[b, pl.ds(t0, R)])


@jax.jit
def _pos_add(x, pos_table):
    mesh = plsc.VectorSubcoreMesh(core_axis_name="c", subcore_axis_name="s")
    f = functools.partial(
        pl.kernel,
        mesh=mesh,
        out_type=jax.ShapeDtypeStruct((B, T, D), jnp.float32),
        scratch_types=[
            pltpu.VMEM((R, D), jnp.float32),
            pltpu.VMEM((R, D), jnp.float32),
            pltpu.VMEM((R, D), jnp.float32),
            pltpu.SemaphoreType.DMA,
        ],
    )(_sc_body)
    return f(x, pos_table)


def kernel(x, pos_table):
    return _pos_add(x, pos_table)


# speedup vs baseline: 2.2795x; 2.2795x over previous
import functools

import jax
import jax.numpy as jnp
from jax import lax
from jax.experimental import pallas as pl
from jax.experimental.pallas import tpu as pltpu
from jax.experimental.pallas import tpu_sc as plsc

B, T, D = 4, 2048, 1024
NW = 32
TW = T // NW
R = 16
NC_CHUNKS = TW // R
K = NC_CHUNKS * B
CD = D // 16


def _sc_body(x_hbm, tab_hbm, out_hbm,
             tb0, tb1, xb0, xb1, ob0, ob1,
             st0, st1, sx0, sx1, so0, so1):
    xbuf = (xb0, xb1)
    tbuf = (tb0, tb1)
    obuf = (ob0, ob1)
    sem_x = (sx0, sx1)
    sem_t = (st0, st1)
    sem_o = (so0, so1)

    wid = lax.axis_index("s") * 2 + lax.axis_index("c")
    t_base = wid * TW

    hx = [None] * K
    ho = [None] * K
    ht = [None] * NC_CHUNKS

    def start_x(k):
        b, ci = k % B, k // B
        t0 = t_base + ci * R
        hx[k] = pltpu.async_copy(x_hbm.at[b, pl.ds(t0, R)], xbuf[k % 2],
                                 sem_x[k % 2])

    def start_t(ci):
        t0 = t_base + ci * R
        ht[ci] = pltpu.async_copy(tab_hbm.at[pl.ds(t0, R)], tbuf[ci % 2],
                                  sem_t[ci % 2])

    start_t(0)
    start_t(1)
    start_x(0)
    start_x(1)

    for k in range(K):
        p = k % 2
        ci = k // B
        b = k % B
        q = ci % 2
        hx[k].wait()
        if b == 0:
            ht[ci].wait()
        if k >= 2:
            ho[k - 2].wait()
        xb, tb, ob = xbuf[p], tbuf[q], obuf[p]

        @plsc.parallel_loop(0, R * CD, unroll=8)
        def add(i):
            r = i // CD
            off = pl.ds((i % CD) * 16, 16)
            ob[r, off] = xb[r, off] + tb[r, off]

        t0 = t_base + ci * R
        ho[k] = pltpu.async_copy(obuf[p], out_hbm.at[b, pl.ds(t0, R)],
                                 sem_o[p])
        if k + 2 < K:
            start_x(k + 2)
        if b == B - 1 and ci + 2 < NC_CHUNKS:
            start_t(ci + 2)

    ho[K - 2].wait()
    ho[K - 1].wait()


@jax.jit
def _pos_add(x, pos_table):
    mesh = plsc.VectorSubcoreMesh(core_axis_name="c", subcore_axis_name="s")
    f = functools.partial(
        pl.kernel,
        mesh=mesh,
        out_type=jax.ShapeDtypeStruct((B, T, D), jnp.float32),
        scratch_types=[
            pltpu.VMEM((R, D), jnp.float32),
            pltpu.VMEM((R, D), jnp.float32),
            pltpu.VMEM((R, D), jnp.float32),
            pltpu.VMEM((R, D), jnp.float32),
            pltpu.VMEM((R, D), jnp.float32),
            pltpu.VMEM((R, D), jnp.float32),
            pltpu.SemaphoreType.DMA,
            pltpu.SemaphoreType.DMA,
            pltpu.SemaphoreType.DMA,
            pltpu.SemaphoreType.DMA,
            pltpu.SemaphoreType.DMA,
            pltpu.SemaphoreType.DMA,
        ],
    )(_sc_body)
    return f(x, pos_table)


def kernel(x, pos_table):
    return _pos_add(x, pos_table)
